# baseline (speedup 1.0000x reference)
import functools

import jax
import jax.numpy as jnp
from jax import lax
from jax.experimental import pallas as pl
from jax.experimental.pallas import tpu as pltpu
from jax.experimental.pallas import tpu_sc as plsc

N = 10000
E = 320000
D = 128
D_COND = 16
B = 8

NC, NS = 2, 16
NW = NC * NS
EPT = E // NW
CHUNK = 200
GRP = 100
GPC = CHUNK // GRP
NCHUNK = EPT // CHUNK
NIDX = EPT // GRP
RPT = 624
TAIL = N - NS * RPT


def _sc_scatter_add(row3d, attr3d, zrows):
    mesh = plsc.VectorSubcoreMesh(core_axis_name="c", subcore_axis_name="s")

    @functools.partial(
        pl.kernel,
        out_type=jax.ShapeDtypeStruct((NC, N, D), jnp.float32),
        mesh=mesh,
        scratch_types=[
            pltpu.VMEM((NIDX, GRP), jnp.int32),
            pltpu.VMEM((CHUNK, D), jnp.float32),
            pltpu.VMEM_SHARED((N, D), jnp.float32),
            pltpu.SemaphoreType.DMA,
        ],
    )
    def scatter_kernel(row_hbm, attr_hbm, z_hbm, out_hbm, idx_v, attr_v,
                       acc_sh, isem):
        cid = lax.axis_index("c")
        sid = lax.axis_index("s")
        wid = cid * NS + sid
        rbase = sid * RPT
        idx_cp = pltpu.async_copy(row_hbm.at[wid], idx_v, isem)
        pltpu.sync_copy(z_hbm.at[pl.ds(0, RPT)], acc_sh.at[pl.ds(rbase, RPT)])

        @pl.when(sid == 0)
        def _zero_tail():
            pltpu.sync_copy(z_hbm.at[pl.ds(0, TAIL)],
                            acc_sh.at[pl.ds(NS * RPT, TAIL)])

        idx_cp.wait()
        plsc.subcore_barrier()

        ebase = wid * EPT
        pltpu.async_copy(attr_hbm.at[pl.ds(ebase, CHUNK)], attr_v, isem)

        def body(c, carry):
            pltpu.make_async_copy(
                attr_hbm.at[pl.ds(ebase + c * CHUNK, CHUNK)], attr_v,
                isem).wait()
            for j in range(GPC):
                pltpu.sync_copy(
                    attr_v.at[pl.ds(j * GRP, GRP)],
                    acc_sh.at[idx_v.at[c * GPC + j]],
                    add=True,
                )

            @pl.when(c + 1 < NCHUNK)
            def _refill():
                pltpu.async_copy(
                    attr_hbm.at[pl.ds(ebase + (c + 1) * CHUNK, CHUNK)],
                    attr_v, isem)

            return carry

        lax.fori_loop(0, NCHUNK, body, 0)
        plsc.subcore_barrier()
        pltpu.sync_copy(
            acc_sh.at[pl.ds(rbase, RPT)],
            out_hbm.at[cid, pl.ds(rbase, RPT)],
        )

        @pl.when(sid == 0)
        def _copy_tail():
            pltpu.sync_copy(
                acc_sh.at[pl.ds(NS * RPT, TAIL)],
                out_hbm.at[cid, pl.ds(NS * RPT, TAIL)],
            )

    return scatter_kernel(row3d, attr3d, zrows)


def _tc_mlp(x, parts, batch2d, u, W1, b1, W2, b2):
    R = 1000

    def body(x_ref, p_ref, bt_ref, u_ref, W1_ref, b1_ref, W2_ref,
             b2_ref, o_ref):
        agg = p_ref[0] + p_ref[1]
        oh = (bt_ref[...] == lax.broadcasted_iota(jnp.int32, (1, B), 1)
              ).astype(jnp.float32)
        uw = jnp.dot(u_ref[...], W1_ref[2 * D:2 * D + D_COND, :],
                     preferred_element_type=jnp.float32)
        z = (jnp.dot(x_ref[...], W1_ref[0:D, :],
                     preferred_element_type=jnp.float32)
             + jnp.dot(agg, W1_ref[D:2 * D, :],
                       preferred_element_type=jnp.float32)
             + jnp.dot(oh, uw, preferred_element_type=jnp.float32)
             + b1_ref[...])
        h = jnp.maximum(z, 0.0)
        o_ref[...] = jnp.dot(h, W2_ref[...],
                             preferred_element_type=jnp.float32) + b2_ref[...]

    return pl.pallas_call(
        body,
        grid=(N // R,),
        in_specs=[
            pl.BlockSpec((R, D), lambda i: (i, 0)),
            pl.BlockSpec((NC, R, D), lambda i: (0, i, 0)),
            pl.BlockSpec((R, 1), lambda i: (i, 0)),
            pl.BlockSpec((B, D_COND), lambda i: (0, 0)),
            pl.BlockSpec((2 * D + D_COND, D), lambda i: (0, 0)),
            pl.BlockSpec((1, D), lambda i: (0, 0)),
            pl.BlockSpec((D, D), lambda i: (0, 0)),
            pl.BlockSpec((1, D), lambda i: (0, 0)),
        ],
        out_specs=pl.BlockSpec((R, D), lambda i: (i, 0)),
        out_shape=jax.ShapeDtypeStruct((N, D), jnp.float32),
    )(x, parts, batch2d, u, W1, b1, W2, b2)


def kernel(x, edge_index, edge_attr, u, batch, W1, b1, W2, b2):
    row = edge_index[0].astype(jnp.int32)
    row3d = row.reshape(NW, NIDX, GRP)
    attr3d = edge_attr
    zrows = jnp.zeros((RPT, D), jnp.float32)
    parts = _sc_scatter_add(row3d, attr3d, zrows)
    return _tc_mlp(
        x, parts,
        batch.astype(jnp.int32).reshape(N, 1), u,
        W1, b1.reshape(1, D), W2, b2.reshape(1, D),
    )


# speedup vs baseline: 1.0051x; 1.0051x over previous
import functools

import jax
import jax.numpy as jnp
from jax import lax
from jax.experimental import pallas as pl
from jax.experimental.pallas import tpu as pltpu
from jax.experimental.pallas import tpu_sc as plsc

N = 10000
E = 320000
D = 128
D_COND = 16
B = 8

NC, NS = 2, 16
NW = NC * NS
EPT = E // NW
CHUNK = 200
GRP = 100
GPC = CHUNK // GRP
NCHUNK = EPT // CHUNK
NIDX = EPT // GRP
RPT = 624
TAIL = N - NS * RPT


def _sc_scatter_add(row3d, attr3d, zrows):
    mesh = plsc.VectorSubcoreMesh(core_axis_name="c", subcore_axis_name="s")

    @functools.partial(
        pl.kernel,
        out_type=jax.ShapeDtypeStruct((NC, N, D), jnp.float32),
        mesh=mesh,
        scratch_types=[
            pltpu.VMEM((NIDX, GRP), jnp.int32),
            pltpu.VMEM((CHUNK, D), jnp.float32),
            pltpu.VMEM_SHARED((N, D), jnp.float32),
            pltpu.SemaphoreType.DMA,
        ],
    )
    def scatter_kernel(row_hbm, attr_hbm, z_hbm, out_hbm, idx_v, attr_v,
                       acc_sh, isem):
        cid = lax.axis_index("c")
        sid = lax.axis_index("s")
        wid = cid * NS + sid
        rbase = sid * RPT
        idx_cp = pltpu.async_copy(row_hbm.at[wid], idx_v, isem)
        pltpu.sync_copy(z_hbm.at[pl.ds(0, RPT)], acc_sh.at[pl.ds(rbase, RPT)])

        @pl.when(sid == 0)
        def _zero_tail():
            pltpu.sync_copy(z_hbm.at[pl.ds(0, TAIL)],
                            acc_sh.at[pl.ds(NS * RPT, TAIL)])

        idx_cp.wait()
        plsc.subcore_barrier()

        ebase = wid * EPT

        def body(c, carry):
            pltpu.sync_copy(
                attr_hbm.at[pl.ds(ebase + c * CHUNK, CHUNK)], attr_v)
            for j in range(GPC):
                pltpu.sync_copy(
                    attr_v.at[pl.ds(j * GRP, GRP)],
                    acc_sh.at[idx_v.at[c * GPC + j]],
                    add=True,
                )
            return carry

        lax.fori_loop(0, NCHUNK, body, 0)
        plsc.subcore_barrier()
        pltpu.sync_copy(
            acc_sh.at[pl.ds(rbase, RPT)],
            out_hbm.at[cid, pl.ds(rbase, RPT)],
        )

        @pl.when(sid == 0)
        def _copy_tail():
            pltpu.sync_copy(
                acc_sh.at[pl.ds(NS * RPT, TAIL)],
                out_hbm.at[cid, pl.ds(NS * RPT, TAIL)],
            )

    return scatter_kernel(row3d, attr3d, zrows)


def _tc_mlp(x, parts, batch2d, u, W1, b1, W2, b2):
    R = 1000

    def body(x_ref, p_ref, bt_ref, u_ref, W1_ref, b1_ref, W2_ref,
             b2_ref, o_ref):
        agg = p_ref[0] + p_ref[1]
        oh = (bt_ref[...] == lax.broadcasted_iota(jnp.int32, (1, B), 1)
              ).astype(jnp.float32)
        uw = jnp.dot(u_ref[...], W1_ref[2 * D:2 * D + D_COND, :],
                     preferred_element_type=jnp.float32)
        z = (jnp.dot(x_ref[...], W1_ref[0:D, :],
                     preferred_element_type=jnp.float32)
             + jnp.dot(agg, W1_ref[D:2 * D, :],
                       preferred_element_type=jnp.float32)
             + jnp.dot(oh, uw, preferred_element_type=jnp.float32)
             + b1_ref[...])
        h = jnp.maximum(z, 0.0)
        o_ref[...] = jnp.dot(h, W2_ref[...],
                             preferred_element_type=jnp.float32) + b2_ref[...]

    return pl.pallas_call(
        body,
        grid=(N // R,),
        in_specs=[
            pl.BlockSpec((R, D), lambda i: (i, 0)),
            pl.BlockSpec((NC, R, D), lambda i: (0, i, 0)),
            pl.BlockSpec((R, 1), lambda i: (i, 0)),
            pl.BlockSpec((B, D_COND), lambda i: (0, 0)),
            pl.BlockSpec((2 * D + D_COND, D), lambda i: (0, 0)),
            pl.BlockSpec((1, D), lambda i: (0, 0)),
            pl.BlockSpec((D, D), lambda i: (0, 0)),
            pl.BlockSpec((1, D), lambda i: (0, 0)),
        ],
        out_specs=pl.BlockSpec((R, D), lambda i: (i, 0)),
        out_shape=jax.ShapeDtypeStruct((N, D), jnp.float32),
    )(x, parts, batch2d, u, W1, b1, W2, b2)


def kernel(x, edge_index, edge_attr, u, batch, W1, b1, W2, b2):
    row = edge_index[0].astype(jnp.int32)
    row3d = row.reshape(NW, NIDX, GRP)
    attr3d = edge_attr
    zrows = jnp.zeros((RPT, D), jnp.float32)
    parts = _sc_scatter_add(row3d, attr3d, zrows)
    return _tc_mlp(
        x, parts,
        batch.astype(jnp.int32).reshape(N, 1), u,
        W1, b1.reshape(1, D), W2, b2.reshape(1, D),
    )
